# Initial kernel scaffold; baseline (speedup 1.0000x reference)
import jax
import jax.numpy as jnp
from jax.experimental import pallas as pl


def kernel(stem1_w, stem1_scale, stem1_bias, stem2_w, stem2_scale, stem2_bias, layer3_w, layer3_scale, layer3_bias, layer4_w, layer4_scale, layer4_bias, aspp0_w, aspp0_scale, aspp0_bias, aspp1_w, aspp1_scale, aspp1_bias, aspp2_w, aspp2_scale, aspp2_bias, aspp3_w, aspp3_scale, aspp3_bias, aspp_pool_w, aspp_pool_scale, aspp_pool_bias, aspp_proj_w, aspp_proj_scale, aspp_proj_bias, dec_low_w, dec_low_scale, dec_low_bias, dec_conv1_w, dec_conv1_scale, dec_conv1_bias, dec_conv2_w, dec_conv2_scale, dec_conv2_bias, classifier_w, classifier_b, x):
    raise NotImplementedError("write your pallas kernel here")


# speedup vs baseline: 1.0499x; 1.0499x over previous
import functools

import jax
import jax.numpy as jnp
import numpy as np
from jax.experimental import pallas as pl
from jax.experimental.pallas import tpu as pltpu

_BF = jnp.bfloat16
_F32 = jnp.float32


def _rup(x, m):
    return ((x + m - 1) // m) * m


def _tile(m, target, align=8):
    t = min(target, m)
    t -= t % align
    while t >= align:
        if m % t == 0:
            return t
        t -= align
    return m


def _interp_mat(out_size, in_size):
    if out_size == 1 or in_size == 1:
        m = np.zeros((out_size, in_size), np.float32)
        m[:, 0] = 1.0
        return m
    src = np.arange(out_size, dtype=np.float64) * (in_size - 1) / (out_size - 1)
    i0 = np.clip(np.floor(src).astype(np.int64), 0, in_size - 1)
    i1 = np.clip(i0 + 1, 0, in_size - 1)
    w1 = (src - i0).astype(np.float32)
    w0 = 1.0 - w1
    m = np.zeros((out_size, in_size), np.float32)
    m[np.arange(out_size), i0] += w0
    m[np.arange(out_size), i1] += w1
    return m


def _mm_body(*refs, n_in, relu, n_out):
    a_refs = refs[:n_in]
    b_refs = refs[n_in:2 * n_in]
    c_ref = refs[2 * n_in]
    o_ref = refs[2 * n_in + 1]
    acc = jnp.dot(a_refs[0][...], b_refs[0][...], preferred_element_type=_F32)
    for i in range(1, n_in):
        acc = acc + jnp.dot(a_refs[i][...], b_refs[i][...],
                            preferred_element_type=_F32)
    acc = acc + c_ref[...]
    if relu:
        acc = jnp.maximum(acc, 0.0)
    if n_out != acc.shape[-1]:
        acc = acc[:, :n_out]
    o_ref[...] = acc.astype(o_ref.dtype)


def _mm(a_list, b_list, bias, relu, tile_m, out_dtype):
    m = a_list[0].shape[0]
    n = b_list[0].shape[1]
    np_ = _rup(n, 128)
    if bias is None:
        bias_p = jnp.zeros((1, np_), _F32)
    else:
        bias_p = jnp.pad(bias.astype(_F32).reshape(1, n), ((0, 0), (0, np_ - n)))
    a_bf = [a.astype(_BF) for a in a_list]
    b_bf = [jnp.pad(b.astype(_F32), ((0, 0), (0, np_ - n))).astype(_BF)
            for b in b_list]
    tm = _tile(m, tile_m)
    n_in = len(a_list)
    in_specs = (
        [pl.BlockSpec((tm, a.shape[1]), lambda i: (i, 0)) for a in a_bf]
        + [pl.BlockSpec((b.shape[0], np_), lambda i: (0, 0)) for b in b_bf]
        + [pl.BlockSpec((1, np_), lambda i: (0, 0))]
    )
    return pl.pallas_call(
        functools.partial(_mm_body, n_in=n_in, relu=relu, n_out=n),
        out_shape=jax.ShapeDtypeStruct((m, n), out_dtype),
        grid=(m // tm,),
        in_specs=in_specs,
        out_specs=pl.BlockSpec((tm, n), lambda i: (i, 0)),
        compiler_params=pltpu.CompilerParams(
            dimension_semantics=("parallel",),
            vmem_limit_bytes=64 * 1024 * 1024,
        ),
    )(*a_bf, *b_bf, bias_p)


def _colmm_body(a_ref, b_ref, o_ref):
    o_ref[...] = jnp.dot(a_ref[...], b_ref[...],
                         preferred_element_type=_F32).astype(o_ref.dtype)


def _col_mm(a, b, tile_n, out_dtype):
    m, k = a.shape
    n = b.shape[1]
    tn = _tile(n, tile_n, align=128)
    return pl.pallas_call(
        _colmm_body,
        out_shape=jax.ShapeDtypeStruct((m, n), out_dtype),
        grid=(n // tn,),
        in_specs=[pl.BlockSpec((m, k), lambda j: (0, 0)),
                  pl.BlockSpec((k, tn), lambda j: (0, j))],
        out_specs=pl.BlockSpec((m, tn), lambda j: (0, j)),
        compiler_params=pltpu.CompilerParams(
            dimension_semantics=("parallel",),
            vmem_limit_bytes=64 * 1024 * 1024,
        ),
    )(a.astype(_BF), b.astype(_BF))


def _mm2_body(a_ref, b1_ref, c1_ref, b2_ref, c2_ref, o_ref, *, n_out):
    acc = jnp.dot(a_ref[...], b1_ref[...], preferred_element_type=_F32)
    acc = jnp.maximum(acc + c1_ref[...], 0.0).astype(_BF)
    acc2 = jnp.dot(acc, b2_ref[...], preferred_element_type=_F32) + c2_ref[...]
    o_ref[...] = acc2[:, :n_out].astype(o_ref.dtype)


def _mm2(a, b1, c1, b2, c2, tile_m, out_dtype):
    m, k = a.shape
    n1 = b1.shape[1]
    n1p = _rup(n1, 128)
    n2 = b2.shape[1]
    n2p = _rup(n2, 128)
    b1p = jnp.pad(b1.astype(_F32), ((0, 0), (0, n1p - n1))).astype(_BF)
    c1p = jnp.pad(c1.astype(_F32).reshape(1, n1), ((0, 0), (0, n1p - n1)))
    b2p = jnp.pad(b2.astype(_F32), ((0, n1p - b2.shape[0]), (0, n2p - n2))).astype(_BF)
    c2p = jnp.pad(c2.astype(_F32).reshape(1, n2), ((0, 0), (0, n2p - n2)))
    tm = _tile(m, tile_m)
    return pl.pallas_call(
        functools.partial(_mm2_body, n_out=n2),
        out_shape=jax.ShapeDtypeStruct((m, n2), out_dtype),
        grid=(m // tm,),
        in_specs=[pl.BlockSpec((tm, k), lambda i: (i, 0)),
                  pl.BlockSpec((k, n1p), lambda i: (0, 0)),
                  pl.BlockSpec((1, n1p), lambda i: (0, 0)),
                  pl.BlockSpec((n1p, n2p), lambda i: (0, 0)),
                  pl.BlockSpec((1, n2p), lambda i: (0, 0))],
        out_specs=pl.BlockSpec((tm, n2), lambda i: (i, 0)),
        compiler_params=pltpu.CompilerParams(
            dimension_semantics=("parallel",),
            vmem_limit_bytes=64 * 1024 * 1024,
        ),
    )(a.astype(_BF), b1p, c1p, b2p, c2p)


def _aspp_body(h_ref, p1_ref, w0_ref, w1_ref, w2_ref, w3_ref, wp_ref,
               j0_ref, j1_ref, j2_ref, j3_ref, j4_ref,
               pm_ref, em_ref,
               c0_ref, c1_ref, c2_ref, c3_ref, cp_ref, cj_ref, o_ref):
    h = h_ref[...]
    b0 = jnp.maximum(jnp.dot(h, w0_ref[...], preferred_element_type=_F32)
                     + c0_ref[...], 0.0).astype(_BF)
    b1 = jnp.maximum(jnp.dot(p1_ref[...], w1_ref[...], preferred_element_type=_F32)
                     + c1_ref[...], 0.0).astype(_BF)
    b2 = jnp.maximum(jnp.dot(h, w2_ref[...], preferred_element_type=_F32)
                     + c2_ref[...], 0.0).astype(_BF)
    b3 = jnp.maximum(jnp.dot(h, w3_ref[...], preferred_element_type=_F32)
                     + c3_ref[...], 0.0).astype(_BF)
    acc = jnp.dot(b0, j0_ref[...], preferred_element_type=_F32)
    acc = acc + jnp.dot(b1, j1_ref[...], preferred_element_type=_F32)
    acc = acc + jnp.dot(b2, j2_ref[...], preferred_element_type=_F32)
    acc = acc + jnp.dot(b3, j3_ref[...], preferred_element_type=_F32)
    pm = jnp.dot(pm_ref[...], h, preferred_element_type=_F32)
    b4 = jnp.maximum(jnp.dot(pm.astype(_BF), wp_ref[...],
                             preferred_element_type=_F32) + cp_ref[...], 0.0)
    c4 = jnp.dot(b4.astype(_BF), j4_ref[...], preferred_element_type=_F32)
    acc = acc + jnp.dot(em_ref[...], c4.astype(_BF), preferred_element_type=_F32)
    acc = jnp.maximum(acc + cj_ref[...], 0.0)
    o_ref[...] = acc[:, :o_ref.shape[-1]].astype(o_ref.dtype)


def _aspp(h2d, p1, w0, w1, w2, w3, wp, wj_parts, biases, n_img, cout):
    m = h2d.shape[0]
    g = 2 if n_img % 2 == 0 else 1
    tm = m // g
    c0, c1, c2, c3, cp, cj = [jnp.pad(b.astype(_F32).reshape(1, -1),
                                      ((0, 0), (0, 128 - b.shape[0])))
                              for b in biases]
    pool_mat = jnp.asarray(np.kron(np.eye(n_img, dtype=np.float32),
                                   np.full((1, 64), 1.0 / 64, np.float32))).astype(_BF)
    exp_mat = jnp.asarray(np.kron(np.eye(n_img, dtype=np.float32),
                                  np.ones((64, 1), np.float32))).astype(_BF)

    def wpad(w):
        return jnp.pad(w.astype(_F32), ((0, 0), (0, 128 - w.shape[1]))).astype(_BF)

    def jpad(w):
        return jnp.pad(w.astype(_F32),
                       ((0, 128 - w.shape[0]), (0, 128 - w.shape[1]))).astype(_BF)

    in_specs = [
        pl.BlockSpec((tm, 32), lambda i: (i, 0)),
        pl.BlockSpec((tm, p1.shape[1]), lambda i: (i, 0)),
        pl.BlockSpec((32, 128), lambda i: (0, 0)),
        pl.BlockSpec((p1.shape[1], 128), lambda i: (0, 0)),
        pl.BlockSpec((32, 128), lambda i: (0, 0)),
        pl.BlockSpec((32, 128), lambda i: (0, 0)),
        pl.BlockSpec((32, 128), lambda i: (0, 0)),
    ] + [pl.BlockSpec((128, 128), lambda i: (0, 0))] * 5 + [
        pl.BlockSpec((n_img, tm), lambda i: (0, i)),
        pl.BlockSpec((tm, n_img), lambda i: (i, 0)),
    ] + [pl.BlockSpec((1, 128), lambda i: (0, 0))] * 6
    return pl.pallas_call(
        _aspp_body,
        out_shape=jax.ShapeDtypeStruct((m, cout), _BF),
        grid=(g,),
        in_specs=in_specs,
        out_specs=pl.BlockSpec((tm, cout), lambda i: (i, 0)),
        compiler_params=pltpu.CompilerParams(
            dimension_semantics=("parallel",),
            vmem_limit_bytes=64 * 1024 * 1024,
        ),
    )(h2d, p1, wpad(w0), wpad(w1), wpad(w2), wpad(w3), wpad(wp),
      *[jpad(w) for w in wj_parts], pool_mat, exp_mat, c0, c1, c2, c3, cp, cj)


def _im2col(x, kh, kw, stride, dilation, padding):
    n, h, w, c = x.shape
    hp, wp = h + 2 * padding, w + 2 * padding
    ho = (hp - dilation * (kh - 1) - 1) // stride + 1
    wo = (wp - dilation * (kw - 1) - 1) // stride + 1
    xp = jnp.pad(x, ((0, 0), (padding, padding), (padding, padding), (0, 0)))
    cols = []
    for ih in range(kh):
        for iw in range(kw):
            h0 = ih * dilation
            w0 = iw * dilation
            cols.append(xp[:, h0:h0 + (ho - 1) * stride + 1:stride,
                           w0:w0 + (wo - 1) * stride + 1:stride, :])
    return jnp.concatenate(cols, axis=-1).reshape(n * ho * wo, kh * kw * c), ho, wo


def _fold(w, scale):
    wm = w.reshape(-1, w.shape[-1]).astype(_F32)
    if scale is not None:
        wm = wm * scale[None, :]
    return wm


def _conv(x, w, scale, bias, stride=1, padding=0, relu=True, tile_m=4096):
    kh, kw, _, cout = w.shape
    n = x.shape[0]
    if kh == 1 and kw == 1 and stride == 1 and padding == 0:
        patches = x.reshape(-1, x.shape[-1])
        ho, wo = x.shape[1], x.shape[2]
    else:
        patches, ho, wo = _im2col(x, kh, kw, stride, 1, padding)
    y = _mm([patches], [_fold(w, scale)], bias, relu, tile_m, _BF)
    return y.reshape(n, ho, wo, cout)


def kernel(stem1_w, stem1_scale, stem1_bias, stem2_w, stem2_scale, stem2_bias,
           layer3_w, layer3_scale, layer3_bias, layer4_w, layer4_scale,
           layer4_bias, aspp0_w, aspp0_scale, aspp0_bias, aspp1_w, aspp1_scale,
           aspp1_bias, aspp2_w, aspp2_scale, aspp2_bias, aspp3_w, aspp3_scale,
           aspp3_bias, aspp_pool_w, aspp_pool_scale, aspp_pool_bias,
           aspp_proj_w, aspp_proj_scale, aspp_proj_bias, dec_low_w,
           dec_low_scale, dec_low_bias, dec_conv1_w, dec_conv1_scale,
           dec_conv1_bias, dec_conv2_w, dec_conv2_scale, dec_conv2_bias,
           classifier_w, classifier_b, x):
    n, _, s, _ = x.shape
    xh = jnp.transpose(x, (0, 2, 3, 1)).astype(_BF)

    h = _conv(xh, stem1_w, stem1_scale, stem1_bias, stride=2, padding=1)
    low = _conv(h, stem2_w, stem2_scale, stem2_bias, stride=2, padding=1)
    h = _conv(low, layer3_w, layer3_scale, layer3_bias, stride=2, padding=1)
    h = _conv(h, layer4_w, layer4_scale, layer4_bias, stride=2, padding=1)
    sf = s // 16

    p1, _, _ = _im2col(h, 3, 3, 1, 6, 6)
    h2d = h.reshape(-1, h.shape[-1])
    wj = _fold(aspp_proj_w, aspp_proj_scale)
    wj_parts = [wj[16 * i:16 * (i + 1), :] for i in range(5)]
    ha = _aspp(
        h2d, p1,
        _fold(aspp0_w, aspp0_scale),
        _fold(aspp1_w, aspp1_scale),
        _fold(aspp2_w[1:2, 1:2], aspp2_scale),
        _fold(aspp3_w[1:2, 1:2], aspp3_scale),
        _fold(aspp_pool_w, aspp_pool_scale),
        wj_parts,
        [aspp0_bias, aspp1_bias, aspp2_bias, aspp3_bias, aspp_pool_bias,
         aspp_proj_bias],
        n, wj.shape[1])

    low_f = _conv(low, dec_low_w, dec_low_scale, dec_low_bias)
    sd = low.shape[1]
    r2d = jnp.asarray(np.kron(_interp_mat(sd, sf), _interp_mat(sd, sf)))
    hm = jnp.transpose(ha.reshape(n, sf * sf, -1), (1, 0, 2)).reshape(sf * sf, -1)
    hu = _col_mm(r2d, hm, 768, _BF)
    hu = jnp.transpose(hu.reshape(sd, sd, n, -1), (2, 0, 1, 3))

    p_h, _, _ = _im2col(hu, 3, 3, 1, 1, 1)
    p_l, _, _ = _im2col(low_f, 3, 3, 1, 1, 1)
    w1 = _fold(dec_conv1_w, dec_conv1_scale)
    w1t = w1.reshape(9, -1, w1.shape[1])
    d1 = _mm([p_h, p_l],
             [w1t[:, :hu.shape[-1], :].reshape(-1, w1.shape[1]),
              w1t[:, hu.shape[-1]:, :].reshape(-1, w1.shape[1])],
             dec_conv1_bias, True, 4096, _BF)
    d1 = d1.reshape(n, sd, sd, -1)

    p2, _, _ = _im2col(d1, 3, 3, 1, 1, 1)
    cls = _mm2(p2, _fold(dec_conv2_w, dec_conv2_scale), dec_conv2_bias,
               classifier_w.reshape(classifier_w.shape[2], classifier_w.shape[3]),
               classifier_b, 4096, _BF)
    nc = cls.shape[-1]

    rh = jnp.asarray(_interp_mat(s, sd))
    xt = jnp.transpose(cls.reshape(n, sd, sd, nc), (1, 0, 2, 3)).reshape(sd, -1)
    t1 = _col_mm(rh, xt, 8192, _BF)
    x2 = jnp.transpose(t1.reshape(s, n, sd, nc), (1, 3, 0, 2)).reshape(-1, sd)
    out = _mm([x2], [jnp.asarray(_interp_mat(s, sd)).T], None, False, 4096, _F32)
    return out.reshape(n, nc, s, s)


# speedup vs baseline: 15.7760x; 15.0267x over previous
import functools

import jax
import jax.numpy as jnp
import numpy as np
from jax.experimental import pallas as pl
from jax.experimental.pallas import tpu as pltpu

_BF = jnp.bfloat16
_F32 = jnp.float32


def _rup(x, m):
    return ((x + m - 1) // m) * m


def _tile(m, target, align=8):
    t = min(target, m)
    t -= t % align
    while t >= align:
        if m % t == 0:
            return t
        t -= align
    return m


def _interp_mat(out_size, in_size):
    if out_size == 1 or in_size == 1:
        m = np.zeros((out_size, in_size), np.float32)
        m[:, 0] = 1.0
        return m
    src = np.arange(out_size, dtype=np.float64) * (in_size - 1) / (out_size - 1)
    i0 = np.clip(np.floor(src).astype(np.int64), 0, in_size - 1)
    i1 = np.clip(i0 + 1, 0, in_size - 1)
    w1 = (src - i0).astype(np.float32)
    w0 = 1.0 - w1
    m = np.zeros((out_size, in_size), np.float32)
    m[np.arange(out_size), i0] += w0
    m[np.arange(out_size), i1] += w1
    return m


def _mm_body(*refs, n_in, relu, n_out):
    a_refs = refs[:n_in]
    b_refs = refs[n_in:2 * n_in]
    c_ref = refs[2 * n_in]
    o_ref = refs[2 * n_in + 1]
    acc = jnp.dot(a_refs[0][...], b_refs[0][...], preferred_element_type=_F32)
    for i in range(1, n_in):
        acc = acc + jnp.dot(a_refs[i][...], b_refs[i][...],
                            preferred_element_type=_F32)
    acc = acc + c_ref[...]
    if relu:
        acc = jnp.maximum(acc, 0.0)
    if n_out != acc.shape[-1]:
        acc = acc[:, :n_out]
    o_ref[...] = acc.astype(o_ref.dtype)


def _mm(a_list, b_list, bias, relu, tile_m, out_dtype):
    m = a_list[0].shape[0]
    n = b_list[0].shape[1]
    np_ = _rup(n, 128)
    if bias is None:
        bias_p = jnp.zeros((1, np_), _F32)
    else:
        bias_p = jnp.pad(bias.astype(_F32).reshape(1, n), ((0, 0), (0, np_ - n)))
    a_bf = [a.astype(_BF) for a in a_list]
    b_bf = [jnp.pad(b.astype(_F32), ((0, 0), (0, np_ - n))).astype(_BF)
            for b in b_list]
    tm = _tile(m, tile_m)
    n_in = len(a_list)
    in_specs = (
        [pl.BlockSpec((tm, a.shape[1]), lambda i: (i, 0)) for a in a_bf]
        + [pl.BlockSpec((b.shape[0], np_), lambda i: (0, 0)) for b in b_bf]
        + [pl.BlockSpec((1, np_), lambda i: (0, 0))]
    )
    return pl.pallas_call(
        functools.partial(_mm_body, n_in=n_in, relu=relu, n_out=n),
        out_shape=jax.ShapeDtypeStruct((m, n), out_dtype),
        grid=(m // tm,),
        in_specs=in_specs,
        out_specs=pl.BlockSpec((tm, n), lambda i: (i, 0)),
        compiler_params=pltpu.CompilerParams(
            dimension_semantics=("parallel",),
            vmem_limit_bytes=64 * 1024 * 1024,
        ),
    )(*a_bf, *b_bf, bias_p)


def _colmm_body(a_ref, b_ref, o_ref):
    o_ref[...] = jnp.dot(a_ref[...], b_ref[...],
                         preferred_element_type=_F32).astype(o_ref.dtype)


def _col_mm(a, b, tile_n, out_dtype):
    m, k = a.shape
    n = b.shape[1]
    tn = _tile(n, tile_n, align=128)
    return pl.pallas_call(
        _colmm_body,
        out_shape=jax.ShapeDtypeStruct((m, n), out_dtype),
        grid=(n // tn,),
        in_specs=[pl.BlockSpec((m, k), lambda j: (0, 0)),
                  pl.BlockSpec((k, tn), lambda j: (0, j))],
        out_specs=pl.BlockSpec((m, tn), lambda j: (0, j)),
        compiler_params=pltpu.CompilerParams(
            dimension_semantics=("parallel",),
            vmem_limit_bytes=64 * 1024 * 1024,
        ),
    )(a.astype(_BF), b.astype(_BF))


def _mm2_body(a_ref, b1_ref, c1_ref, b2_ref, c2_ref, o_ref, *, n_out):
    acc = jnp.dot(a_ref[...], b1_ref[...], preferred_element_type=_F32)
    acc = jnp.maximum(acc + c1_ref[...], 0.0).astype(_BF)
    acc2 = jnp.dot(acc, b2_ref[...], preferred_element_type=_F32) + c2_ref[...]
    o_ref[...] = acc2[:, :n_out].astype(o_ref.dtype)


def _mm2(a, b1, c1, b2, c2, tile_m, out_dtype):
    m, k = a.shape
    n1 = b1.shape[1]
    n1p = _rup(n1, 128)
    n2 = b2.shape[1]
    n2p = _rup(n2, 128)
    b1p = jnp.pad(b1.astype(_F32), ((0, 0), (0, n1p - n1))).astype(_BF)
    c1p = jnp.pad(c1.astype(_F32).reshape(1, n1), ((0, 0), (0, n1p - n1)))
    b2p = jnp.pad(b2.astype(_F32), ((0, n1p - b2.shape[0]), (0, n2p - n2))).astype(_BF)
    c2p = jnp.pad(c2.astype(_F32).reshape(1, n2), ((0, 0), (0, n2p - n2)))
    tm = _tile(m, tile_m)
    return pl.pallas_call(
        functools.partial(_mm2_body, n_out=n2),
        out_shape=jax.ShapeDtypeStruct((m, n2), out_dtype),
        grid=(m // tm,),
        in_specs=[pl.BlockSpec((tm, k), lambda i: (i, 0)),
                  pl.BlockSpec((k, n1p), lambda i: (0, 0)),
                  pl.BlockSpec((1, n1p), lambda i: (0, 0)),
                  pl.BlockSpec((n1p, n2p), lambda i: (0, 0)),
                  pl.BlockSpec((1, n2p), lambda i: (0, 0))],
        out_specs=pl.BlockSpec((tm, n2), lambda i: (i, 0)),
        compiler_params=pltpu.CompilerParams(
            dimension_semantics=("parallel",),
            vmem_limit_bytes=64 * 1024 * 1024,
        ),
    )(a.astype(_BF), b1p, c1p, b2p, c2p)


def _aspp_body(h_ref, p1_ref, w0_ref, w1_ref, w2_ref, w3_ref, wp_ref,
               j0_ref, j1_ref, j2_ref, j3_ref, j4_ref,
               pm_ref, em_ref,
               c0_ref, c1_ref, c2_ref, c3_ref, cp_ref, cj_ref, o_ref):
    h = h_ref[...]
    b0 = jnp.maximum(jnp.dot(h, w0_ref[...], preferred_element_type=_F32)
                     + c0_ref[...], 0.0).astype(_BF)
    b1 = jnp.maximum(jnp.dot(p1_ref[...], w1_ref[...], preferred_element_type=_F32)
                     + c1_ref[...], 0.0).astype(_BF)
    b2 = jnp.maximum(jnp.dot(h, w2_ref[...], preferred_element_type=_F32)
                     + c2_ref[...], 0.0).astype(_BF)
    b3 = jnp.maximum(jnp.dot(h, w3_ref[...], preferred_element_type=_F32)
                     + c3_ref[...], 0.0).astype(_BF)
    acc = jnp.dot(b0, j0_ref[...], preferred_element_type=_F32)
    acc = acc + jnp.dot(b1, j1_ref[...], preferred_element_type=_F32)
    acc = acc + jnp.dot(b2, j2_ref[...], preferred_element_type=_F32)
    acc = acc + jnp.dot(b3, j3_ref[...], preferred_element_type=_F32)
    pm = jnp.dot(pm_ref[...], h, preferred_element_type=_F32)
    b4 = jnp.maximum(jnp.dot(pm.astype(_BF), wp_ref[...],
                             preferred_element_type=_F32) + cp_ref[...], 0.0)
    c4 = jnp.dot(b4.astype(_BF), j4_ref[...], preferred_element_type=_F32)
    acc = acc + jnp.dot(em_ref[...], c4.astype(_BF), preferred_element_type=_F32)
    acc = jnp.maximum(acc + cj_ref[...], 0.0)
    o_ref[...] = acc[:, :o_ref.shape[-1]].astype(o_ref.dtype)


def _aspp(h2d, p1, w0, w1, w2, w3, wp, wj_parts, biases, n_img, cout):
    m = h2d.shape[0]
    g = 2 if n_img % 2 == 0 else 1
    tm = m // g
    c0, c1, c2, c3, cp, cj = [jnp.pad(b.astype(_F32).reshape(1, -1),
                                      ((0, 0), (0, 128 - b.shape[0])))
                              for b in biases]
    pool_mat = jnp.asarray(np.kron(np.eye(n_img, dtype=np.float32),
                                   np.full((1, 64), 1.0 / 64, np.float32))).astype(_BF)
    exp_mat = jnp.asarray(np.kron(np.eye(n_img, dtype=np.float32),
                                  np.ones((64, 1), np.float32))).astype(_BF)

    def wpad(w):
        return jnp.pad(w.astype(_F32), ((0, 0), (0, 128 - w.shape[1]))).astype(_BF)

    def jpad(w):
        return jnp.pad(w.astype(_F32),
                       ((0, 128 - w.shape[0]), (0, 128 - w.shape[1]))).astype(_BF)

    in_specs = [
        pl.BlockSpec((tm, 32), lambda i: (i, 0)),
        pl.BlockSpec((tm, p1.shape[1]), lambda i: (i, 0)),
        pl.BlockSpec((32, 128), lambda i: (0, 0)),
        pl.BlockSpec((p1.shape[1], 128), lambda i: (0, 0)),
        pl.BlockSpec((32, 128), lambda i: (0, 0)),
        pl.BlockSpec((32, 128), lambda i: (0, 0)),
        pl.BlockSpec((32, 128), lambda i: (0, 0)),
    ] + [pl.BlockSpec((128, 128), lambda i: (0, 0))] * 5 + [
        pl.BlockSpec((n_img, tm), lambda i: (0, i)),
        pl.BlockSpec((tm, n_img), lambda i: (i, 0)),
    ] + [pl.BlockSpec((1, 128), lambda i: (0, 0))] * 6
    return pl.pallas_call(
        _aspp_body,
        out_shape=jax.ShapeDtypeStruct((m, cout), _BF),
        grid=(g,),
        in_specs=in_specs,
        out_specs=pl.BlockSpec((tm, cout), lambda i: (i, 0)),
        compiler_params=pltpu.CompilerParams(
            dimension_semantics=("parallel",),
            vmem_limit_bytes=64 * 1024 * 1024,
        ),
    )(h2d, p1, wpad(w0), wpad(w1), wpad(w2), wpad(w3), wpad(wp),
      *[jpad(w) for w in wj_parts], pool_mat, exp_mat, c0, c1, c2, c3, cp, cj)


def _s2conv_body(x_ref, w_ref, b_ref, o_ref, *, ho, kp):
    nb = o_ref.shape[0]
    xs = x_ref[...]
    acc = None
    for kh in range(3):
        if kh == 0:
            a = xs[:, 0:ho, 0:kp]
        elif kh == 1:
            a = xs[:, 0:ho, kp:2 * kp]
        else:
            a = xs[:, 1:ho + 1, 0:kp]
        d = jnp.dot(a.reshape(nb * ho, kp), w_ref[kh],
                    preferred_element_type=_F32)
        acc = d if acc is None else acc + d
    acc = jnp.maximum(acc + b_ref[...], 0.0)
    o_ref[...] = acc.reshape(nb, ho, acc.shape[-1]).astype(o_ref.dtype)


def _s2conv(x3, w, scale, bias, wi, cin, cout):
    n, h, _ = x3.shape
    ho, wo = h // 2, wi // 2
    hp, wp = h + 2, wi + 2
    wpc = wp * cin
    kp = _rup(wpc, 128)
    xp = jnp.pad(x3, ((0, 0), (1, 1), (cin, kp - wpc + cin)))
    xp = xp.reshape(n, hp // 2, 2 * kp)
    msel = np.zeros((3, wp, wo), np.float32)
    for kw in range(3):
        msel[kw, kw + 2 * np.arange(wo), np.arange(wo)] = 1.0
    wf = w.astype(_F32)
    if scale is not None:
        wf = wf * scale[None, None, None, :]
    wbig = jnp.stack([
        jnp.einsum('qwc,qio->wico', jnp.asarray(msel), wf[kh]).reshape(wpc, wo * cout)
        for kh in range(3)
    ])
    wbig = jnp.pad(wbig, ((0, 0), (0, kp - wpc), (0, 0))).astype(_BF)
    bt = jnp.tile(bias.astype(_F32), wo).reshape(1, wo * cout)
    nb = min(max(128 // ho, 1), n)
    while n % nb:
        nb -= 1
    return pl.pallas_call(
        functools.partial(_s2conv_body, ho=ho, kp=kp),
        out_shape=jax.ShapeDtypeStruct((n, ho, wo * cout), _BF),
        grid=(n // nb,),
        in_specs=[pl.BlockSpec((nb, hp // 2, 2 * kp), lambda i: (i, 0, 0)),
                  pl.BlockSpec((3, kp, wo * cout), lambda i: (0, 0, 0)),
                  pl.BlockSpec((1, wo * cout), lambda i: (0, 0))],
        out_specs=pl.BlockSpec((nb, ho, wo * cout), lambda i: (i, 0, 0)),
        compiler_params=pltpu.CompilerParams(
            dimension_semantics=("parallel",),
            vmem_limit_bytes=64 * 1024 * 1024,
        ),
    )(xp, wbig, bt)


def _im2col(x, kh, kw, stride, dilation, padding):
    n, h, w, c = x.shape
    hp, wp = h + 2 * padding, w + 2 * padding
    ho = (hp - dilation * (kh - 1) - 1) // stride + 1
    wo = (wp - dilation * (kw - 1) - 1) // stride + 1
    xp = jnp.pad(x, ((0, 0), (padding, padding), (padding, padding), (0, 0)))
    cols = []
    for ih in range(kh):
        for iw in range(kw):
            h0 = ih * dilation
            w0 = iw * dilation
            cols.append(xp[:, h0:h0 + (ho - 1) * stride + 1:stride,
                           w0:w0 + (wo - 1) * stride + 1:stride, :])
    return jnp.concatenate(cols, axis=-1).reshape(n * ho * wo, kh * kw * c), ho, wo


def _fold(w, scale):
    wm = w.reshape(-1, w.shape[-1]).astype(_F32)
    if scale is not None:
        wm = wm * scale[None, :]
    return wm


def _conv(x, w, scale, bias, stride=1, padding=0, relu=True, tile_m=4096):
    kh, kw, _, cout = w.shape
    n = x.shape[0]
    if kh == 1 and kw == 1 and stride == 1 and padding == 0:
        patches = x.reshape(-1, x.shape[-1])
        ho, wo = x.shape[1], x.shape[2]
    else:
        patches, ho, wo = _im2col(x, kh, kw, stride, 1, padding)
    y = _mm([patches], [_fold(w, scale)], bias, relu, tile_m, _BF)
    return y.reshape(n, ho, wo, cout)


def kernel(stem1_w, stem1_scale, stem1_bias, stem2_w, stem2_scale, stem2_bias,
           layer3_w, layer3_scale, layer3_bias, layer4_w, layer4_scale,
           layer4_bias, aspp0_w, aspp0_scale, aspp0_bias, aspp1_w, aspp1_scale,
           aspp1_bias, aspp2_w, aspp2_scale, aspp2_bias, aspp3_w, aspp3_scale,
           aspp3_bias, aspp_pool_w, aspp_pool_scale, aspp_pool_bias,
           aspp_proj_w, aspp_proj_scale, aspp_proj_bias, dec_low_w,
           dec_low_scale, dec_low_bias, dec_conv1_w, dec_conv1_scale,
           dec_conv1_bias, dec_conv2_w, dec_conv2_scale, dec_conv2_bias,
           classifier_w, classifier_b, x):
    n, _, s, _ = x.shape
    xh = jnp.transpose(x, (0, 2, 3, 1)).astype(_BF)

    h1 = _s2conv(xh.reshape(n, s, s * 3), stem1_w, stem1_scale, stem1_bias,
                 s, 3, 8)
    h2 = _s2conv(h1, stem2_w, stem2_scale, stem2_bias, s // 2, 8, 16)
    h3 = _s2conv(h2, layer3_w, layer3_scale, layer3_bias, s // 4, 16, 24)
    h4 = _s2conv(h3, layer4_w, layer4_scale, layer4_bias, s // 8, 24, 32)
    sf = s // 16
    h = h4.reshape(n, sf, sf, 32)
    low = h2.reshape(n, s // 4, s // 4, 16)

    p1, _, _ = _im2col(h, 3, 3, 1, 6, 6)
    h2d = h.reshape(-1, h.shape[-1])
    wj = _fold(aspp_proj_w, aspp_proj_scale)
    wj_parts = [wj[16 * i:16 * (i + 1), :] for i in range(5)]
    ha = _aspp(
        h2d, p1,
        _fold(aspp0_w, aspp0_scale),
        _fold(aspp1_w, aspp1_scale),
        _fold(aspp2_w[1:2, 1:2], aspp2_scale),
        _fold(aspp3_w[1:2, 1:2], aspp3_scale),
        _fold(aspp_pool_w, aspp_pool_scale),
        wj_parts,
        [aspp0_bias, aspp1_bias, aspp2_bias, aspp3_bias, aspp_pool_bias,
         aspp_proj_bias],
        n, wj.shape[1])

    low_f = _conv(low, dec_low_w, dec_low_scale, dec_low_bias)
    sd = low.shape[1]
    r2d = jnp.asarray(np.kron(_interp_mat(sd, sf), _interp_mat(sd, sf)))
    hm = jnp.transpose(ha.reshape(n, sf * sf, -1), (1, 0, 2)).reshape(sf * sf, -1)
    hu = _col_mm(r2d, hm, 768, _BF)
    hu = jnp.transpose(hu.reshape(sd, sd, n, -1), (2, 0, 1, 3))

    p_h, _, _ = _im2col(hu, 3, 3, 1, 1, 1)
    p_l, _, _ = _im2col(low_f, 3, 3, 1, 1, 1)
    w1 = _fold(dec_conv1_w, dec_conv1_scale)
    w1t = w1.reshape(9, -1, w1.shape[1])
    d1 = _mm([p_h, p_l],
             [w1t[:, :hu.shape[-1], :].reshape(-1, w1.shape[1]),
              w1t[:, hu.shape[-1]:, :].reshape(-1, w1.shape[1])],
             dec_conv1_bias, True, 4096, _BF)
    d1 = d1.reshape(n, sd, sd, -1)

    p2, _, _ = _im2col(d1, 3, 3, 1, 1, 1)
    cls = _mm2(p2, _fold(dec_conv2_w, dec_conv2_scale), dec_conv2_bias,
               classifier_w.reshape(classifier_w.shape[2], classifier_w.shape[3]),
               classifier_b, 4096, _BF)
    nc = cls.shape[-1]

    rh = jnp.asarray(_interp_mat(s, sd))
    xt = jnp.transpose(cls.reshape(n, sd, sd, nc), (1, 0, 2, 3)).reshape(sd, -1)
    t1 = _col_mm(rh, xt, 8192, _BF)
    x2 = jnp.transpose(t1.reshape(s, n, sd, nc), (1, 3, 0, 2)).reshape(-1, sd)
    out = _mm([x2], [jnp.asarray(_interp_mat(s, sd)).T], None, False, 4096, _F32)
    return out.reshape(n, nc, s, s)


# speedup vs baseline: 31.6037x; 2.0033x over previous
import functools

import jax
import jax.numpy as jnp
import numpy as np
from jax.experimental import pallas as pl
from jax.experimental.pallas import tpu as pltpu

_BF = jnp.bfloat16
_F32 = jnp.float32


def _rup(x, m):
    return ((x + m - 1) // m) * m


def _tile(m, target, align=8):
    t = min(target, m)
    t -= t % align
    while t >= align:
        if m % t == 0:
            return t
        t -= align
    return m


def _interp_mat(out_size, in_size):
    if out_size == 1 or in_size == 1:
        m = np.zeros((out_size, in_size), np.float32)
        m[:, 0] = 1.0
        return m
    src = np.arange(out_size, dtype=np.float64) * (in_size - 1) / (out_size - 1)
    i0 = np.clip(np.floor(src).astype(np.int64), 0, in_size - 1)
    i1 = np.clip(i0 + 1, 0, in_size - 1)
    w1 = (src - i0).astype(np.float32)
    w0 = 1.0 - w1
    m = np.zeros((out_size, in_size), np.float32)
    m[np.arange(out_size), i0] += w0
    m[np.arange(out_size), i1] += w1
    return m


def _cparams():
    return pltpu.CompilerParams(
        dimension_semantics=("parallel",),
        vmem_limit_bytes=64 * 1024 * 1024,
    )


def _kron_eye(w2d, blocks):
    k, n = w2d.shape
    eye = jnp.asarray(np.eye(blocks, dtype=np.float32))
    return jnp.einsum('pq,io->piqo', eye, w2d.astype(_F32)).reshape(
        blocks * k, blocks * n).astype(_BF)


def _fold(w, scale):
    wf = w.astype(_F32)
    if scale is not None:
        wf = wf * scale[None, None, None, :]
    return wf


def _btile(bias, blocks):
    return jnp.tile(bias.astype(_F32), blocks).reshape(1, -1)


def _wsel(wf_kh, wp, wo, stride, dil):
    cin, cout = wf_kh.shape[1], wf_kh.shape[2]
    msel = np.zeros((3, wp, wo), np.float32)
    for kw in range(3):
        cols = np.arange(wo)
        msel[kw, cols * stride + kw * dil, cols] = 1.0
    return jnp.einsum('qwc,qio->wico', jnp.asarray(msel), wf_kh).reshape(
        wp * cin, wo * cout)


def _s2conv_body(x_ref, w_ref, b_ref, o_ref, *, ho, kp):
    nb = o_ref.shape[0]
    xs = x_ref[...]
    acc = None
    for kh in range(3):
        if kh == 0:
            a = xs[:, 0:ho, 0:kp]
        elif kh == 1:
            a = xs[:, 0:ho, kp:2 * kp]
        else:
            a = xs[:, 1:ho + 1, 0:kp]
        d = jnp.dot(a.reshape(nb * ho, kp), w_ref[kh],
                    preferred_element_type=_F32)
        acc = d if acc is None else acc + d
    acc = jnp.maximum(acc + b_ref[...], 0.0)
    o_ref[...] = acc.reshape(nb, ho, acc.shape[-1]).astype(o_ref.dtype)


def _s2conv(x3, w, scale, bias, wi, cin, cout):
    n, h, _ = x3.shape
    ho, wo = h // 2, wi // 2
    hp, wp = h + 2, wi + 2
    wpc = wp * cin
    kp = _rup(wpc, 128)
    xp = jnp.pad(x3, ((0, 0), (1, 1), (cin, kp - wpc + cin)))
    xp = xp.reshape(n, hp // 2, 2 * kp)
    wf = _fold(w, scale)
    wbig = jnp.stack([_wsel(wf[kh], wp, wo, 2, 1) for kh in range(3)])
    wbig = jnp.pad(wbig, ((0, 0), (0, kp - wpc), (0, 0))).astype(_BF)
    bt = _btile(bias, wo)
    nb = min(max(128 // ho, 1), n)
    while n % nb:
        nb -= 1
    return pl.pallas_call(
        functools.partial(_s2conv_body, ho=ho, kp=kp),
        out_shape=jax.ShapeDtypeStruct((n, ho, wo * cout), _BF),
        grid=(n // nb,),
        in_specs=[pl.BlockSpec((nb, hp // 2, 2 * kp), lambda i: (i, 0, 0)),
                  pl.BlockSpec((3, kp, wo * cout), lambda i: (0, 0, 0)),
                  pl.BlockSpec((1, wo * cout), lambda i: (0, 0))],
        out_specs=pl.BlockSpec((nb, ho, wo * cout), lambda i: (i, 0, 0)),
        compiler_params=_cparams(),
    )(xp, wbig, bt)


def _s1pad(x3, wi, c):
    wpc = (wi + 2) * c
    kp = _rup(wpc, 128)
    return jnp.pad(x3, ((0, 0), (1, 1), (c, kp - wpc + c))), kp


def _dec_body(*refs, n_in, ho, kps, chain):
    x_refs = refs[:n_in]
    w_refs = refs[n_in:2 * n_in]
    b_ref = refs[2 * n_in]
    extra = refs[2 * n_in + 1:]
    nb = extra[-1].shape[0]
    acc = None
    for j in range(n_in):
        xs = x_refs[j][...]
        for kh in range(3):
            a = xs[:, kh:kh + ho, :].reshape(nb * ho, kps[j])
            d = jnp.dot(a, w_refs[j][kh], preferred_element_type=_F32)
            acc = d if acc is None else acc + d
    acc = jnp.maximum(acc + b_ref[...], 0.0)
    if chain:
        wc_ref, bc_ref, o_ref = extra
        acc2 = jnp.dot(acc.astype(_BF), wc_ref[...],
                       preferred_element_type=_F32) + bc_ref[...]
        o_ref[...] = acc2.reshape(nb, ho, acc2.shape[-1]).astype(o_ref.dtype)
    else:
        o_ref = extra[0]
        o_ref[...] = acc.reshape(nb, ho, acc.shape[-1]).astype(o_ref.dtype)


def _dec_conv(x3_list, cins, wf, bias, wi, cout, chain_w=None, chain_b=None):
    n, ho = x3_list[0].shape[0], x3_list[0].shape[1]
    xps, kps, wbigs = [], [], []
    off = 0
    for x3, cin in zip(x3_list, cins):
        xp, kp = _s1pad(x3, wi, cin)
        wfj = wf[:, :, off:off + cin, :]
        off += cin
        wb = jnp.stack([_wsel(wfj[kh], wi + 2, wi, 1, 1) for kh in range(3)])
        wb = jnp.pad(wb, ((0, 0), (0, kp - (wi + 2) * cin), (0, 0))).astype(_BF)
        xps.append(xp)
        kps.append(kp)
        wbigs.append(wb)
    bt = _btile(bias, wi)
    n_out = wi * cout
    chain = chain_w is not None
    if chain:
        ncls = chain_w.shape[1]
        wc = _kron_eye(chain_w, wi)
        bc = _btile(chain_b, wi)
        n_out = wi * ncls
    nb = min(max(128 // ho, 1), n)
    while n % nb:
        nb -= 1
    in_specs = (
        [pl.BlockSpec((nb, ho + 2, kp), lambda i: (i, 0, 0)) for kp in kps]
        + [pl.BlockSpec((3, kp, wi * cout), lambda i: (0, 0, 0)) for kp in kps]
        + [pl.BlockSpec((1, wi * cout), lambda i: (0, 0))]
    )
    ops = list(xps) + wbigs + [bt]
    if chain:
        in_specs += [pl.BlockSpec((wi * cout, n_out), lambda i: (0, 0)),
                     pl.BlockSpec((1, n_out), lambda i: (0, 0))]
        ops += [wc, bc]
    return pl.pallas_call(
        functools.partial(_dec_body, n_in=len(x3_list), ho=ho,
                          kps=tuple(kps), chain=chain),
        out_shape=jax.ShapeDtypeStruct((n, ho, n_out), _BF),
        grid=(n // nb,),
        in_specs=in_specs,
        out_specs=pl.BlockSpec((nb, ho, n_out), lambda i: (i, 0, 0)),
        compiler_params=_cparams(),
    )(*ops)


def _flat1_body(x_ref, w_ref, b_ref, o_ref):
    nb, ho, kp = x_ref.shape
    a = x_ref[...].reshape(nb * ho, kp)
    acc = jnp.maximum(jnp.dot(a, w_ref[...], preferred_element_type=_F32)
                      + b_ref[...], 0.0)
    o_ref[...] = acc.reshape(nb, ho, acc.shape[-1]).astype(o_ref.dtype)


def _flat1(x3, w2d, bias, wi):
    n, ho, _ = x3.shape
    wk = _kron_eye(w2d, wi)
    bt = _btile(bias, wi)
    n_out = wk.shape[1]
    nb = min(max(256 // ho, 1), n)
    while n % nb:
        nb -= 1
    return pl.pallas_call(
        _flat1_body,
        out_shape=jax.ShapeDtypeStruct((n, ho, n_out), _BF),
        grid=(n // nb,),
        in_specs=[pl.BlockSpec((nb, ho, x3.shape[2]), lambda i: (i, 0, 0)),
                  pl.BlockSpec((wk.shape[0], n_out), lambda i: (0, 0)),
                  pl.BlockSpec((1, n_out), lambda i: (0, 0))],
        out_specs=pl.BlockSpec((nb, ho, n_out), lambda i: (i, 0, 0)),
        compiler_params=_cparams(),
    )(x3, wk, bt)


def _aspp_body(h_ref, hp6_ref, w0_ref, w2_ref, w3_ref, wb1_ref, wp_ref,
               j0_ref, j1_ref, j2_ref, j3_ref, j4_ref, k8_ref,
               p2_ref, c8_ref, e2_ref,
               c0_ref, c1_ref, c2_ref, c3_ref, cp_ref, cj_ref, o_ref):
    nb, sf, lanes = o_ref.shape
    h = h_ref[...]
    b0 = jnp.maximum(jnp.dot(h, w0_ref[...], preferred_element_type=_F32)
                     + c0_ref[...], 0.0).astype(_BF)
    b2 = jnp.maximum(jnp.dot(h, w2_ref[...], preferred_element_type=_F32)
                     + c2_ref[...], 0.0).astype(_BF)
    b3 = jnp.maximum(jnp.dot(h, w3_ref[...], preferred_element_type=_F32)
                     + c3_ref[...], 0.0).astype(_BF)
    hp = hp6_ref[...]
    b1 = None
    for kh in range(3):
        a = hp[:, 6 * kh:6 * kh + sf, :].reshape(nb * sf, hp.shape[-1])
        d = jnp.dot(a, wb1_ref[kh], preferred_element_type=_F32)
        b1 = d if b1 is None else b1 + d
    b1 = jnp.maximum(b1 + c1_ref[...], 0.0).astype(_BF)
    acc = jnp.dot(b0, j0_ref[...], preferred_element_type=_F32)
    acc = acc + jnp.dot(b1, j1_ref[...], preferred_element_type=_F32)
    acc = acc + jnp.dot(b2, j2_ref[...], preferred_element_type=_F32)
    acc = acc + jnp.dot(b3, j3_ref[...], preferred_element_type=_F32)
    pr = jnp.dot(p2_ref[...], h, preferred_element_type=_F32)
    pm = jnp.dot(pr.astype(_BF), c8_ref[...], preferred_element_type=_F32)
    b4 = jnp.maximum(jnp.dot(pm.astype(_BF), wp_ref[...],
                             preferred_element_type=_F32) + cp_ref[...], 0.0)
    c4 = jnp.dot(b4.astype(_BF), j4_ref[...], preferred_element_type=_F32)
    c4t = jnp.dot(c4.astype(_BF), k8_ref[...], preferred_element_type=_F32)
    acc = acc + jnp.dot(e2_ref[...], c4t.astype(_BF),
                        preferred_element_type=_F32)
    acc = jnp.maximum(acc + cj_ref[...], 0.0)
    o_ref[...] = acc.reshape(nb, sf, lanes).astype(o_ref.dtype)


def _aspp(h4, w0, w1, w2, w3, wp, wj, biases, sf, cm, co):
    n = h4.shape[0]
    hflat = h4.reshape(n * sf, sf * cm)
    hp6 = jnp.pad(h4, ((0, 0), (6, 6), (6 * cm, 6 * cm)))
    g = 2 if n % 2 == 0 else 1
    nb = n // g
    wf1 = _fold(w1, None)
    wb1 = jnp.stack([_wsel(wf1[kh], sf + 12, sf, 1, 6) for kh in range(3)])
    wb1 = wb1.astype(_BF)
    k8 = np.zeros((co * sf, co * sf), np.float32)
    for wi_ in range(sf):
        k8[0:co, wi_ * co:(wi_ + 1) * co] = np.eye(co)
    p2 = np.kron(np.eye(n, dtype=np.float32), np.full((1, sf), 1.0 / sf))
    c8 = np.kron(np.full((sf, 1), 1.0 / sf, np.float32), np.eye(cm))
    e2 = np.kron(np.eye(n, dtype=np.float32), np.ones((sf, 1), np.float32))
    c0, c1, c2, c3 = [_btile(b, sf) for b in biases[:4]]
    cp = jnp.pad(biases[4].astype(_F32).reshape(1, -1),
                 ((0, 0), (0, co * sf - co)))
    cj = _btile(biases[5], sf)
    jpads = [_kron_eye(w, sf) for w in wj[:4]] + [
        jnp.pad(wj[4].astype(_F32), ((0, co * sf - wj[4].shape[0]),
                                     (0, co * sf - wj[4].shape[1]))).astype(_BF)]
    wpp = jnp.pad(wp.astype(_F32), ((0, 0), (0, co * sf - co))).astype(_BF)
    lanes = sf * co
    in_specs = [
        pl.BlockSpec((nb * sf, sf * cm), lambda i: (i, 0)),
        pl.BlockSpec((nb, sf + 12, hp6.shape[2]), lambda i: (i, 0, 0)),
        pl.BlockSpec((sf * cm, lanes), lambda i: (0, 0)),
        pl.BlockSpec((sf * cm, lanes), lambda i: (0, 0)),
        pl.BlockSpec((sf * cm, lanes), lambda i: (0, 0)),
        pl.BlockSpec((3, hp6.shape[2], lanes), lambda i: (0, 0, 0)),
        pl.BlockSpec((cm, lanes), lambda i: (0, 0)),
    ] + [pl.BlockSpec((lanes, lanes), lambda i: (0, 0))] * 6 + [
        pl.BlockSpec((n, nb * sf), lambda i: (0, i)),
        pl.BlockSpec((sf * cm, cm), lambda i: (0, 0)),
        pl.BlockSpec((nb * sf, n), lambda i: (i, 0)),
    ] + [pl.BlockSpec((1, lanes), lambda i: (0, 0))] * 6
    return pl.pallas_call(
        _aspp_body,
        out_shape=jax.ShapeDtypeStruct((n, sf, lanes), _BF),
        grid=(g,),
        in_specs=in_specs,
        out_specs=pl.BlockSpec((nb, sf, lanes), lambda i: (i, 0, 0)),
        compiler_params=_cparams(),
    )(hflat, hp6,
      _kron_eye(w0, sf), _kron_eye(w2, sf), _kron_eye(w3, sf), wb1, wpp,
      *jpads, jnp.asarray(k8).astype(_BF),
      jnp.asarray(p2).astype(_BF), jnp.asarray(c8).astype(_BF),
      jnp.asarray(e2).astype(_BF),
      c0, c1, c2, c3, cp, cj)


def _up_body(x_ref, ww_ref, rh_ref, o_ref):
    nb, ho, lanes = o_ref.shape
    sf = x_ref.shape[1]
    xm = jnp.dot(x_ref[...].reshape(nb * sf, x_ref.shape[2]), ww_ref[...],
                 preferred_element_type=_F32)
    hu = jnp.dot(rh_ref[...], xm.astype(_BF), preferred_element_type=_F32)
    o_ref[...] = hu.reshape(nb, ho, lanes).astype(o_ref.dtype)


def _up832(x3, sf, sd, co):
    n = x3.shape[0]
    r1 = _interp_mat(sd, sf)
    ww = np.einsum('ow,ij->wioj', r1, np.eye(co, dtype=np.float32))
    ww = jnp.asarray(ww.reshape(sf * co, sd * co)).astype(_BF)
    bigrh = jnp.asarray(np.kron(np.eye(n, dtype=np.float32), r1)).astype(_BF)
    g = 2 if n % 2 == 0 else 1
    nb = n // g
    return pl.pallas_call(
        _up_body,
        out_shape=jax.ShapeDtypeStruct((n, sd, sd * co), _BF),
        grid=(g,),
        in_specs=[pl.BlockSpec((nb, sf, sf * co), lambda i: (i, 0, 0)),
                  pl.BlockSpec((sf * co, sd * co), lambda i: (0, 0)),
                  pl.BlockSpec((nb * sd, nb * sf), lambda i: (i, i))],
        out_specs=pl.BlockSpec((nb, sd, sd * co), lambda i: (i, 0, 0)),
        compiler_params=_cparams(),
    )(x3, ww, bigrh)


def _mm_body(a_ref, b_ref, o_ref):
    o_ref[...] = jnp.dot(a_ref[...], b_ref[...],
                         preferred_element_type=_F32).astype(o_ref.dtype)


def _mm(a, b, tile_m, out_dtype):
    m, k = a.shape
    n = b.shape[1]
    tm = _tile(m, tile_m)
    return pl.pallas_call(
        _mm_body,
        out_shape=jax.ShapeDtypeStruct((m, n), out_dtype),
        grid=(m // tm,),
        in_specs=[pl.BlockSpec((tm, k), lambda i: (i, 0)),
                  pl.BlockSpec((k, n), lambda i: (0, 0))],
        out_specs=pl.BlockSpec((tm, n), lambda i: (i, 0)),
        compiler_params=_cparams(),
    )(a.astype(_BF), b.astype(_BF))


def _col_mm(a, b, tile_n, out_dtype):
    m, k = a.shape
    n = b.shape[1]
    tn = _tile(n, tile_n, align=128)
    return pl.pallas_call(
        _mm_body,
        out_shape=jax.ShapeDtypeStruct((m, n), out_dtype),
        grid=(n // tn,),
        in_specs=[pl.BlockSpec((m, k), lambda j: (0, 0)),
                  pl.BlockSpec((k, tn), lambda j: (0, j))],
        out_specs=pl.BlockSpec((m, tn), lambda j: (0, j)),
        compiler_params=_cparams(),
    )(a.astype(_BF), b.astype(_BF))


def kernel(stem1_w, stem1_scale, stem1_bias, stem2_w, stem2_scale, stem2_bias,
           layer3_w, layer3_scale, layer3_bias, layer4_w, layer4_scale,
           layer4_bias, aspp0_w, aspp0_scale, aspp0_bias, aspp1_w, aspp1_scale,
           aspp1_bias, aspp2_w, aspp2_scale, aspp2_bias, aspp3_w, aspp3_scale,
           aspp3_bias, aspp_pool_w, aspp_pool_scale, aspp_pool_bias,
           aspp_proj_w, aspp_proj_scale, aspp_proj_bias, dec_low_w,
           dec_low_scale, dec_low_bias, dec_conv1_w, dec_conv1_scale,
           dec_conv1_bias, dec_conv2_w, dec_conv2_scale, dec_conv2_bias,
           classifier_w, classifier_b, x):
    n, _, s, _ = x.shape
    xh = jnp.transpose(x, (0, 2, 3, 1)).astype(_BF).reshape(n, s, s * 3)

    h1 = _s2conv(xh, stem1_w, stem1_scale, stem1_bias, s, 3, 8)
    h2 = _s2conv(h1, stem2_w, stem2_scale, stem2_bias, s // 2, 8, 16)
    h3 = _s2conv(h2, layer3_w, layer3_scale, layer3_bias, s // 4, 16, 24)
    h4 = _s2conv(h3, layer4_w, layer4_scale, layer4_bias, s // 8, 24, 32)
    sf, sd = s // 16, s // 4

    wjf = _fold(aspp_proj_w, aspp_proj_scale).reshape(80, 16)
    ha = _aspp(
        h4,
        _fold(aspp0_w, aspp0_scale).reshape(32, 16),
        _fold(aspp1_w, aspp1_scale),
        _fold(aspp2_w[1:2, 1:2], aspp2_scale).reshape(32, 16),
        _fold(aspp3_w[1:2, 1:2], aspp3_scale).reshape(32, 16),
        _fold(aspp_pool_w, aspp_pool_scale).reshape(32, 16),
        [wjf[16 * i:16 * (i + 1), :] for i in range(5)],
        [aspp0_bias, aspp1_bias, aspp2_bias, aspp3_bias, aspp_pool_bias,
         aspp_proj_bias],
        sf, 32, 16)

    hu = _up832(ha, sf, sd, 16)
    lf = _flat1(h2, _fold(dec_low_w, dec_low_scale).reshape(16, 8),
                dec_low_bias, sd)
    d1 = _dec_conv([hu, lf], [16, 8], _fold(dec_conv1_w, dec_conv1_scale),
                   dec_conv1_bias, sd, 16)
    cls = _dec_conv([d1], [16], _fold(dec_conv2_w, dec_conv2_scale),
                    dec_conv2_bias, sd, 16,
                    chain_w=classifier_w.reshape(16, 21),
                    chain_b=classifier_b)
    nc = 21

    rh = jnp.asarray(_interp_mat(s, sd))
    xt = jnp.transpose(cls, (1, 0, 2)).reshape(sd, n * sd * nc)
    t1 = _col_mm(rh, xt, 8192, _BF)
    x2 = jnp.transpose(t1.reshape(s, n, sd, nc), (1, 3, 0, 2)).reshape(-1, sd)
    out = _mm(x2, jnp.asarray(_interp_mat(s, sd)).T, 4096, _F32)
    return out.reshape(n, nc, s, s)


# speedup vs baseline: 32.6300x; 1.0325x over previous
import functools

import jax
import jax.numpy as jnp
import numpy as np
from jax.experimental import pallas as pl
from jax.experimental.pallas import tpu as pltpu

_BF = jnp.bfloat16
_F32 = jnp.float32


def _rup(x, m):
    return ((x + m - 1) // m) * m


def _tile(m, target, align=8):
    t = min(target, m)
    t -= t % align
    while t >= align:
        if m % t == 0:
            return t
        t -= align
    return m


def _interp_mat(out_size, in_size):
    if out_size == 1 or in_size == 1:
        m = np.zeros((out_size, in_size), np.float32)
        m[:, 0] = 1.0
        return m
    src = np.arange(out_size, dtype=np.float64) * (in_size - 1) / (out_size - 1)
    i0 = np.clip(np.floor(src).astype(np.int64), 0, in_size - 1)
    i1 = np.clip(i0 + 1, 0, in_size - 1)
    w1 = (src - i0).astype(np.float32)
    w0 = 1.0 - w1
    m = np.zeros((out_size, in_size), np.float32)
    m[np.arange(out_size), i0] += w0
    m[np.arange(out_size), i1] += w1
    return m


def _cparams():
    return pltpu.CompilerParams(
        dimension_semantics=("parallel",),
        vmem_limit_bytes=64 * 1024 * 1024,
    )


def _kron_eye(w2d, blocks):
    k, n = w2d.shape
    eye = jnp.asarray(np.eye(blocks, dtype=np.float32))
    return jnp.einsum('pq,io->piqo', eye, w2d.astype(_F32)).reshape(
        blocks * k, blocks * n).astype(_BF)


def _fold(w, scale):
    wf = w.astype(_F32)
    if scale is not None:
        wf = wf * scale[None, None, None, :]
    return wf


def _btile(bias, blocks):
    return jnp.tile(bias.astype(_F32), blocks).reshape(1, -1)


def _wsel3(wf, wp, wo, stride, dil, kp):
    cin, cout = wf.shape[2], wf.shape[3]
    msel = np.zeros((3, wp, wo), np.float32)
    for kw in range(3):
        cols = np.arange(wo)
        msel[kw, cols * stride + kw * dil, cols] = 1.0
    wb = jnp.einsum('qwc,kqio->kwico', jnp.asarray(msel), wf)
    wb = wb.reshape(3, wp * cin, wo * cout)
    return jnp.pad(wb, ((0, 0), (0, kp - wp * cin), (0, 0))).astype(_BF)


def _s2conv_body(x_ref, w_ref, b_ref, o_ref, *, ho, kp):
    nb = o_ref.shape[0]
    xs = x_ref[...]
    acc = None
    for kh in range(3):
        if kh == 0:
            a = xs[:, 0:ho, 0:kp]
        elif kh == 1:
            a = xs[:, 0:ho, kp:2 * kp]
        else:
            a = xs[:, 1:ho + 1, 0:kp]
        d = jnp.dot(a.reshape(nb * ho, kp), w_ref[kh],
                    preferred_element_type=_F32)
        acc = d if acc is None else acc + d
    acc = jnp.maximum(acc + b_ref[...], 0.0)
    o_ref[...] = acc.reshape(nb, ho, acc.shape[-1]).astype(o_ref.dtype)


def _s2conv(x3, w, scale, bias, wi, cin, cout):
    n, h, _ = x3.shape
    ho, wo = h // 2, wi // 2
    hp, wp = h + 2, wi + 2
    wpc = wp * cin
    kp = _rup(wpc, 128)
    xp = jnp.pad(x3, ((0, 0), (1, 1), (cin, kp - wpc + cin)))
    xp = xp.reshape(n, hp // 2, 2 * kp)
    wbig = _wsel3(_fold(w, scale), wp, wo, 2, 1, kp)
    bt = _btile(bias, wo)
    nb = min(max(128 // ho, 1), n)
    while n % nb:
        nb -= 1
    return pl.pallas_call(
        functools.partial(_s2conv_body, ho=ho, kp=kp),
        out_shape=jax.ShapeDtypeStruct((n, ho, wo * cout), _BF),
        grid=(n // nb,),
        in_specs=[pl.BlockSpec((nb, hp // 2, 2 * kp), lambda i: (i, 0, 0)),
                  pl.BlockSpec((3, kp, wo * cout), lambda i: (0, 0, 0)),
                  pl.BlockSpec((1, wo * cout), lambda i: (0, 0))],
        out_specs=pl.BlockSpec((nb, ho, wo * cout), lambda i: (i, 0, 0)),
        compiler_params=_cparams(),
    )(xp, wbig, bt)


def _s1pad(x3, wi, c):
    wpc = (wi + 2) * c
    kp = _rup(wpc, 128)
    return jnp.pad(x3, ((0, 0), (1, 1), (c, kp - wpc + c))), kp


def _dec_body(*refs, n_in, ho, kps, chain):
    x_refs = refs[:n_in]
    w_refs = refs[n_in:2 * n_in]
    b_ref = refs[2 * n_in]
    extra = refs[2 * n_in + 1:]
    nb = extra[-1].shape[0]
    acc = None
    for j in range(n_in):
        xs = x_refs[j][...]
        for kh in range(3):
            a = xs[:, kh:kh + ho, :].reshape(nb * ho, kps[j])
            d = jnp.dot(a, w_refs[j][kh], preferred_element_type=_F32)
            acc = d if acc is None else acc + d
    acc = jnp.maximum(acc + b_ref[...], 0.0)
    if chain:
        wc_ref, bc_ref, o_ref = extra
        acc2 = jnp.dot(acc.astype(_BF), wc_ref[...],
                       preferred_element_type=_F32) + bc_ref[...]
        o_ref[...] = acc2.reshape(nb, ho, acc2.shape[-1]).astype(o_ref.dtype)
    else:
        o_ref = extra[0]
        o_ref[...] = acc.reshape(nb, ho, acc.shape[-1]).astype(o_ref.dtype)


def _dec_conv(x3_list, cins, wf, bias, wi, cout, chain_w=None, chain_b=None):
    n, ho = x3_list[0].shape[0], x3_list[0].shape[1]
    xps, kps, wbigs = [], [], []
    off = 0
    for x3, cin in zip(x3_list, cins):
        xp, kp = _s1pad(x3, wi, cin)
        wfj = wf[:, :, off:off + cin, :]
        off += cin
        wb = _wsel3(wfj, wi + 2, wi, 1, 1, kp)
        xps.append(xp)
        kps.append(kp)
        wbigs.append(wb)
    bt = _btile(bias, wi)
    n_out = wi * cout
    chain = chain_w is not None
    if chain:
        ncls = chain_w.shape[1]
        wc = _kron_eye(chain_w, wi)
        bc = _btile(chain_b, wi)
        n_out = wi * ncls
    nb = min(max(128 // ho, 1), n)
    while n % nb:
        nb -= 1
    in_specs = (
        [pl.BlockSpec((nb, ho + 2, kp), lambda i: (i, 0, 0)) for kp in kps]
        + [pl.BlockSpec((3, kp, wi * cout), lambda i: (0, 0, 0)) for kp in kps]
        + [pl.BlockSpec((1, wi * cout), lambda i: (0, 0))]
    )
    ops = list(xps) + wbigs + [bt]
    if chain:
        in_specs += [pl.BlockSpec((wi * cout, n_out), lambda i: (0, 0)),
                     pl.BlockSpec((1, n_out), lambda i: (0, 0))]
        ops += [wc, bc]
    return pl.pallas_call(
        functools.partial(_dec_body, n_in=len(x3_list), ho=ho,
                          kps=tuple(kps), chain=chain),
        out_shape=jax.ShapeDtypeStruct((n, ho, n_out), _BF),
        grid=(n // nb,),
        in_specs=in_specs,
        out_specs=pl.BlockSpec((nb, ho, n_out), lambda i: (i, 0, 0)),
        compiler_params=_cparams(),
    )(*ops)


def _flat1_body(x_ref, w_ref, b_ref, o_ref):
    nb, ho, kp = x_ref.shape
    a = x_ref[...].reshape(nb * ho, kp)
    acc = jnp.maximum(jnp.dot(a, w_ref[...], preferred_element_type=_F32)
                      + b_ref[...], 0.0)
    o_ref[...] = acc.reshape(nb, ho, acc.shape[-1]).astype(o_ref.dtype)


def _flat1(x3, w2d, bias, wi):
    n, ho, _ = x3.shape
    wk = _kron_eye(w2d, wi)
    bt = _btile(bias, wi)
    n_out = wk.shape[1]
    nb = min(max(256 // ho, 1), n)
    while n % nb:
        nb -= 1
    return pl.pallas_call(
        _flat1_body,
        out_shape=jax.ShapeDtypeStruct((n, ho, n_out), _BF),
        grid=(n // nb,),
        in_specs=[pl.BlockSpec((nb, ho, x3.shape[2]), lambda i: (i, 0, 0)),
                  pl.BlockSpec((wk.shape[0], n_out), lambda i: (0, 0)),
                  pl.BlockSpec((1, n_out), lambda i: (0, 0))],
        out_specs=pl.BlockSpec((nb, ho, n_out), lambda i: (i, 0, 0)),
        compiler_params=_cparams(),
    )(x3, wk, bt)


def _aspp_body(h_ref, hp6_ref, w0_ref, w2_ref, w3_ref, wb1_ref, wp_ref,
               j0_ref, j1_ref, j2_ref, j3_ref, j4_ref, k8_ref,
               p2_ref, c8_ref, e2_ref,
               c0_ref, c1_ref, c2_ref, c3_ref, cp_ref, cj_ref, o_ref):
    nb, sf, lanes = o_ref.shape
    h = h_ref[...]
    b0 = jnp.maximum(jnp.dot(h, w0_ref[...], preferred_element_type=_F32)
                     + c0_ref[...], 0.0).astype(_BF)
    b2 = jnp.maximum(jnp.dot(h, w2_ref[...], preferred_element_type=_F32)
                     + c2_ref[...], 0.0).astype(_BF)
    b3 = jnp.maximum(jnp.dot(h, w3_ref[...], preferred_element_type=_F32)
                     + c3_ref[...], 0.0).astype(_BF)
    hp = hp6_ref[...]
    b1 = None
    for kh in range(3):
        a = hp[:, 6 * kh:6 * kh + sf, :].reshape(nb * sf, hp.shape[-1])
        d = jnp.dot(a, wb1_ref[kh], preferred_element_type=_F32)
        b1 = d if b1 is None else b1 + d
    b1 = jnp.maximum(b1 + c1_ref[...], 0.0).astype(_BF)
    acc = jnp.dot(b0, j0_ref[...], preferred_element_type=_F32)
    acc = acc + jnp.dot(b1, j1_ref[...], preferred_element_type=_F32)
    acc = acc + jnp.dot(b2, j2_ref[...], preferred_element_type=_F32)
    acc = acc + jnp.dot(b3, j3_ref[...], preferred_element_type=_F32)
    pr = jnp.dot(p2_ref[...], h, preferred_element_type=_F32)
    pm = jnp.dot(pr.astype(_BF), c8_ref[...], preferred_element_type=_F32)
    b4 = jnp.maximum(jnp.dot(pm.astype(_BF), wp_ref[...],
                             preferred_element_type=_F32) + cp_ref[...], 0.0)
    c4 = jnp.dot(b4.astype(_BF), j4_ref[...], preferred_element_type=_F32)
    c4t = jnp.dot(c4.astype(_BF), k8_ref[...], preferred_element_type=_F32)
    acc = acc + jnp.dot(e2_ref[...], c4t.astype(_BF),
                        preferred_element_type=_F32)
    acc = jnp.maximum(acc + cj_ref[...], 0.0)
    o_ref[...] = acc.reshape(nb, sf, lanes).astype(o_ref.dtype)


def _aspp(h4, w0, w1, w2, w3, wp, wj, biases, sf, cm, co):
    n = h4.shape[0]
    hflat = h4.reshape(n * sf, sf * cm)
    hp6 = jnp.pad(h4, ((0, 0), (6, 6), (6 * cm, 6 * cm)))
    g = 2 if n % 2 == 0 else 1
    nb = n // g
    wb1 = _wsel3(_fold(w1, None), sf + 12, sf, 1, 6, hp6.shape[2])
    k8 = np.zeros((co * sf, co * sf), np.float32)
    for wi_ in range(sf):
        k8[0:co, wi_ * co:(wi_ + 1) * co] = np.eye(co)
    p2 = np.kron(np.eye(n, dtype=np.float32), np.full((1, sf), 1.0 / sf))
    c8 = np.kron(np.full((sf, 1), 1.0 / sf, np.float32), np.eye(cm))
    e2 = np.kron(np.eye(n, dtype=np.float32), np.ones((sf, 1), np.float32))
    c0, c1, c2, c3 = [_btile(b, sf) for b in biases[:4]]
    cp = jnp.pad(biases[4].astype(_F32).reshape(1, -1),
                 ((0, 0), (0, co * sf - co)))
    cj = _btile(biases[5], sf)
    jpads = [_kron_eye(w, sf) for w in wj[:4]] + [
        jnp.pad(wj[4].astype(_F32), ((0, co * sf - wj[4].shape[0]),
                                     (0, co * sf - wj[4].shape[1]))).astype(_BF)]
    wpp = jnp.pad(wp.astype(_F32), ((0, 0), (0, co * sf - co))).astype(_BF)
    lanes = sf * co
    in_specs = [
        pl.BlockSpec((nb * sf, sf * cm), lambda i: (i, 0)),
        pl.BlockSpec((nb, sf + 12, hp6.shape[2]), lambda i: (i, 0, 0)),
        pl.BlockSpec((sf * cm, lanes), lambda i: (0, 0)),
        pl.BlockSpec((sf * cm, lanes), lambda i: (0, 0)),
        pl.BlockSpec((sf * cm, lanes), lambda i: (0, 0)),
        pl.BlockSpec((3, hp6.shape[2], lanes), lambda i: (0, 0, 0)),
        pl.BlockSpec((cm, lanes), lambda i: (0, 0)),
    ] + [pl.BlockSpec((lanes, lanes), lambda i: (0, 0))] * 6 + [
        pl.BlockSpec((n, nb * sf), lambda i: (0, i)),
        pl.BlockSpec((sf * cm, cm), lambda i: (0, 0)),
        pl.BlockSpec((nb * sf, n), lambda i: (i, 0)),
    ] + [pl.BlockSpec((1, lanes), lambda i: (0, 0))] * 6
    return pl.pallas_call(
        _aspp_body,
        out_shape=jax.ShapeDtypeStruct((n, sf, lanes), _BF),
        grid=(g,),
        in_specs=in_specs,
        out_specs=pl.BlockSpec((nb, sf, lanes), lambda i: (i, 0, 0)),
        compiler_params=_cparams(),
    )(hflat, hp6,
      _kron_eye(w0, sf), _kron_eye(w2, sf), _kron_eye(w3, sf), wb1, wpp,
      *jpads, jnp.asarray(k8).astype(_BF),
      jnp.asarray(p2).astype(_BF), jnp.asarray(c8).astype(_BF),
      jnp.asarray(e2).astype(_BF),
      c0, c1, c2, c3, cp, cj)


def _up_body(x_ref, ww_ref, rh_ref, o_ref):
    nb, ho, lanes = o_ref.shape
    sf = x_ref.shape[1]
    xm = jnp.dot(x_ref[...].reshape(nb * sf, x_ref.shape[2]), ww_ref[...],
                 preferred_element_type=_F32)
    hu = jnp.dot(rh_ref[...], xm.astype(_BF), preferred_element_type=_F32)
    o_ref[...] = hu.reshape(nb, ho, lanes).astype(o_ref.dtype)


def _up832(x3, sf, sd, co):
    n = x3.shape[0]
    r1 = _interp_mat(sd, sf)
    ww = np.einsum('ow,ij->wioj', r1, np.eye(co, dtype=np.float32))
    ww = jnp.asarray(ww.reshape(sf * co, sd * co)).astype(_BF)
    bigrh = jnp.asarray(np.kron(np.eye(n, dtype=np.float32), r1)).astype(_BF)
    g = 2 if n % 2 == 0 else 1
    nb = n // g
    return pl.pallas_call(
        _up_body,
        out_shape=jax.ShapeDtypeStruct((n, sd, sd * co), _BF),
        grid=(g,),
        in_specs=[pl.BlockSpec((nb, sf, sf * co), lambda i: (i, 0, 0)),
                  pl.BlockSpec((sf * co, sd * co), lambda i: (0, 0)),
                  pl.BlockSpec((nb * sd, nb * sf), lambda i: (i, i))],
        out_specs=pl.BlockSpec((nb, sd, sd * co), lambda i: (i, 0, 0)),
        compiler_params=_cparams(),
    )(x3, ww, bigrh)


def _mm_body(a_ref, b_ref, o_ref):
    o_ref[...] = jnp.dot(a_ref[...], b_ref[...],
                         preferred_element_type=_F32).astype(o_ref.dtype)


def _mmT_body(a_ref, b_ref, o_ref):
    o_ref[...] = jax.lax.dot_general(
        a_ref[...], b_ref[...], (((0,), (0,)), ((), ())),
        preferred_element_type=_F32).astype(o_ref.dtype)


def _mmT(at, b, tile_m, out_dtype):
    k, m = at.shape
    n = b.shape[1]
    tm = _tile(m, tile_m, align=128)
    return pl.pallas_call(
        _mmT_body,
        out_shape=jax.ShapeDtypeStruct((m, n), out_dtype),
        grid=(m // tm,),
        in_specs=[pl.BlockSpec((k, tm), lambda i: (0, i)),
                  pl.BlockSpec((k, n), lambda i: (0, 0))],
        out_specs=pl.BlockSpec((tm, n), lambda i: (i, 0)),
        compiler_params=_cparams(),
    )(at.astype(_BF), b.astype(_BF))


def _col_mm(a, b, tile_n, out_dtype):
    m, k = a.shape
    n = b.shape[1]
    tn = _tile(n, tile_n, align=128)
    return pl.pallas_call(
        _mm_body,
        out_shape=jax.ShapeDtypeStruct((m, n), out_dtype),
        grid=(n // tn,),
        in_specs=[pl.BlockSpec((m, k), lambda j: (0, 0)),
                  pl.BlockSpec((k, tn), lambda j: (0, j))],
        out_specs=pl.BlockSpec((m, tn), lambda j: (0, j)),
        compiler_params=_cparams(),
    )(a.astype(_BF), b.astype(_BF))


def kernel(stem1_w, stem1_scale, stem1_bias, stem2_w, stem2_scale, stem2_bias,
           layer3_w, layer3_scale, layer3_bias, layer4_w, layer4_scale,
           layer4_bias, aspp0_w, aspp0_scale, aspp0_bias, aspp1_w, aspp1_scale,
           aspp1_bias, aspp2_w, aspp2_scale, aspp2_bias, aspp3_w, aspp3_scale,
           aspp3_bias, aspp_pool_w, aspp_pool_scale, aspp_pool_bias,
           aspp_proj_w, aspp_proj_scale, aspp_proj_bias, dec_low_w,
           dec_low_scale, dec_low_bias, dec_conv1_w, dec_conv1_scale,
           dec_conv1_bias, dec_conv2_w, dec_conv2_scale, dec_conv2_bias,
           classifier_w, classifier_b, x):
    n, _, s, _ = x.shape
    xh = jnp.transpose(x, (0, 2, 3, 1)).astype(_BF).reshape(n, s, s * 3)

    h1 = _s2conv(xh, stem1_w, stem1_scale, stem1_bias, s, 3, 8)
    h2 = _s2conv(h1, stem2_w, stem2_scale, stem2_bias, s // 2, 8, 16)
    h3 = _s2conv(h2, layer3_w, layer3_scale, layer3_bias, s // 4, 16, 24)
    h4 = _s2conv(h3, layer4_w, layer4_scale, layer4_bias, s // 8, 24, 32)
    sf, sd = s // 16, s // 4

    wjf = _fold(aspp_proj_w, aspp_proj_scale).reshape(80, 16)
    ha = _aspp(
        h4,
        _fold(aspp0_w, aspp0_scale).reshape(32, 16),
        _fold(aspp1_w, aspp1_scale),
        _fold(aspp2_w[1:2, 1:2], aspp2_scale).reshape(32, 16),
        _fold(aspp3_w[1:2, 1:2], aspp3_scale).reshape(32, 16),
        _fold(aspp_pool_w, aspp_pool_scale).reshape(32, 16),
        [wjf[16 * i:16 * (i + 1), :] for i in range(5)],
        [aspp0_bias, aspp1_bias, aspp2_bias, aspp3_bias, aspp_pool_bias,
         aspp_proj_bias],
        sf, 32, 16)

    hu = _up832(ha, sf, sd, 16)
    lf = _flat1(h2, _fold(dec_low_w, dec_low_scale).reshape(16, 8),
                dec_low_bias, sd)
    d1 = _dec_conv([hu, lf], [16, 8], _fold(dec_conv1_w, dec_conv1_scale),
                   dec_conv1_bias, sd, 16)
    cls = _dec_conv([d1], [16], _fold(dec_conv2_w, dec_conv2_scale),
                    dec_conv2_bias, sd, 16,
                    chain_w=classifier_w.reshape(16, 21),
                    chain_b=classifier_b)
    nc = 21

    rh = jnp.asarray(_interp_mat(s, sd))
    xt = jnp.transpose(cls, (1, 0, 2)).reshape(sd, n * sd * nc)
    t1 = _col_mm(rh, xt, 8192, _BF)
    x2t = jnp.transpose(t1.reshape(s, n, sd, nc), (2, 1, 3, 0)).reshape(sd, -1)
    out = _mmT(x2t, jnp.asarray(_interp_mat(s, sd)).T, 4096, _F32)
    return out.reshape(n, nc, s, s)
